# Initial kernel scaffold; baseline (speedup 1.0000x reference)
import jax
import jax.numpy as jnp
from jax.experimental import pallas as pl


def kernel(field_indices, table, W1, b1, W2, b2, W3, b3):
    raise NotImplementedError("write your pallas kernel here")


# speedup vs baseline: 2.1054x; 2.1054x over previous
import functools

import jax
import jax.numpy as jnp
from jax import lax
from jax.experimental import pallas as pl
from jax.experimental.pallas import tpu as pltpu
from jax.experimental.pallas import tpu_sc as plsc

_N_FIELDS = 26
_HASH_SIZE = 100000
_EMBED_DIM = 32
_LANES = 16


def _sc_geometry():
    try:
        info = plsc.get_sparse_core_info()
        return info.num_cores, info.num_subcores
    except Exception:
        return 2, 16


@functools.partial(jax.jit, static_argnums=(2, 3))
def _gather(fi_flat, table, nc, ns):
    total = fi_flat.shape[0]
    nw = nc * ns
    per_w = total // nw
    ch = 128
    n_ch = per_w // ch
    assert per_w % ch == 0 and per_w % _LANES == 0

    mesh = plsc.VectorSubcoreMesh(core_axis_name="c", subcore_axis_name="s",
                                  num_cores=nc, num_subcores=ns)

    @functools.partial(
        pl.kernel,
        out_type=jax.ShapeDtypeStruct((total, _EMBED_DIM), jnp.float32),
        mesh=mesh,
        scratch_types=[
            pltpu.VMEM((per_w,), jnp.int32),
            pltpu.VMEM((per_w, _EMBED_DIM), jnp.float32),
            pltpu.SemaphoreType.DMA,
        ],
        compiler_params=pltpu.CompilerParams(use_tc_tiling_on_sc=False),
    )
    def gather_kernel(fi_hbm, table_hbm, out_hbm, idx_v, rows_v, sem):
        wid = lax.axis_index("s") * nc + lax.axis_index("c")
        base = wid * per_w
        pltpu.sync_copy(fi_hbm.at[pl.ds(base, per_w)], idx_v)

        copies = []
        for j in range(n_ch):
            def body(t, _, j=j):
                start = j * ch + t * _LANES
                pos = start + lax.iota(jnp.int32, _LANES)
                sl = pl.ds(start, _LANES)
                idx_v[sl] = idx_v[sl] + (pos % _N_FIELDS) * _HASH_SIZE
                return 0

            lax.fori_loop(0, ch // _LANES, body, 0)
            copies.append(pltpu.async_copy(
                table_hbm.at[idx_v.at[pl.ds(j * ch, ch)]],
                rows_v.at[pl.ds(j * ch, ch)],
                sem,
            ))
        for c in copies:
            c.wait()
        pltpu.sync_copy(rows_v, out_hbm.at[pl.ds(base, per_w)])

    return gather_kernel(fi_flat, table)


def _mlp_body(x_ref, w1_ref, b1_ref, w2_ref, b2_ref, w3_ref, b3_ref, o_ref):
    x = x_ref[...]
    h = jnp.dot(x, w1_ref[...], preferred_element_type=jnp.float32)
    h = jnp.maximum(h + b1_ref[...], 0.0)
    h = jnp.dot(h, w2_ref[...], preferred_element_type=jnp.float32)
    h = jnp.maximum(h + b2_ref[...], 0.0)
    o_ref[...] = jnp.sum(h * w3_ref[...], axis=1, keepdims=True) + b3_ref[...]


def _mlp(x, w1t, b1, w2t, b2, w3, b3):
    bsz, in_dim = x.shape
    blk = 512
    grid = bsz // blk
    return pl.pallas_call(
        _mlp_body,
        grid=(grid,),
        in_specs=[
            pl.BlockSpec((blk, in_dim), lambda i: (i, 0)),
            pl.BlockSpec(w1t.shape, lambda i: (0, 0)),
            pl.BlockSpec(b1.shape, lambda i: (0, 0)),
            pl.BlockSpec(w2t.shape, lambda i: (0, 0)),
            pl.BlockSpec(b2.shape, lambda i: (0, 0)),
            pl.BlockSpec(w3.shape, lambda i: (0, 0)),
            pl.BlockSpec(b3.shape, lambda i: (0, 0)),
        ],
        out_specs=pl.BlockSpec((blk, 1), lambda i: (i, 0)),
        out_shape=jax.ShapeDtypeStruct((bsz, 1), jnp.float32),
    )(x, w1t, b1, w2t, b2, w3, b3)


def kernel(field_indices, table, W1, b1, W2, b2, W3, b3):
    bsz, nf = field_indices.shape
    nc, ns = _sc_geometry()
    fi_flat = field_indices.reshape(-1).astype(jnp.int32)
    embeds = _gather(fi_flat, table, nc, ns)
    x = embeds.reshape(bsz, nf * _EMBED_DIM)
    out = _mlp(x, W1.T, b1[None, :], W2.T, b2[None, :], W3, b3[None, :])
    return out[:, 0]


# speedup vs baseline: 3.7049x; 1.7597x over previous
import functools

import jax
import jax.numpy as jnp
from jax import lax
from jax.experimental import pallas as pl
from jax.experimental.pallas import tpu as pltpu
from jax.experimental.pallas import tpu_sc as plsc

_N_FIELDS = 26
_HASH_SIZE = 100000
_EMBED_DIM = 32
_LANES = 16


def _sc_geometry():
    try:
        info = plsc.get_sparse_core_info()
        return info.num_cores, info.num_subcores
    except Exception:
        return 2, 16


@functools.partial(jax.jit, static_argnums=(2, 3))
def _gather(fi_flat, table3, nc, ns):
    total = fi_flat.shape[0]
    nw = nc * ns
    per_w = total // nw
    S = 64
    n_stage = per_w // S
    assert per_w % S == 0 and S % _LANES == 0 and per_w % _N_FIELDS == 0

    mesh = plsc.VectorSubcoreMesh(core_axis_name="c", subcore_axis_name="s",
                                  num_cores=nc, num_subcores=ns)

    @functools.partial(
        pl.kernel,
        out_type=jax.ShapeDtypeStruct((total * _EMBED_DIM,), jnp.float32),
        mesh=mesh,
        scratch_types=[
            pltpu.VMEM((per_w,), jnp.int32),
            pltpu.VMEM((per_w,), jnp.int32),
            pltpu.VMEM((S, 8, _EMBED_DIM), jnp.float32),
            pltpu.VMEM((S * _EMBED_DIM,), jnp.float32),
            pltpu.SemaphoreType.DMA,
        ],
        compiler_params=pltpu.CompilerParams(needs_layout_passes=False),
    )
    def gather_kernel(fi_hbm, table_hbm, out_hbm, g_v, gq_v, rows_v, xflat,
                      sem):
        wid = lax.axis_index("s") * nc + lax.axis_index("c")
        base = wid * per_w
        lanes_iota = lax.iota(jnp.int32, _LANES)

        pltpu.sync_copy(fi_hbm.at[pl.ds(base, per_w)], g_v)

        def off(k, _):
            sl = pl.ds(k * _LANES, _LANES)
            pos = k * _LANES + lanes_iota
            g = g_v[sl] + (pos % _N_FIELDS) * _HASH_SIZE
            g_v[sl] = g
            gq_v[sl] = g // 8
            return 0

        lax.fori_loop(0, per_w // _LANES, off, 0)

        def stage(st, _):
            s0 = st * S

            def fire_grp(gi, _):
                ql = gq_v[pl.ds(s0 + gi * _LANES, _LANES)]
                for l in range(_LANES):
                    q = jnp.sum(jnp.where(lanes_iota == l, ql, 0))
                    pltpu.make_async_copy(
                        table_hbm.at[q],
                        rows_v.at[gi * _LANES + l],
                        sem,
                    ).start()
                return 0

            lax.fori_loop(0, S // _LANES, fire_grp, 0)
            pltpu.make_async_copy(
                table_hbm.at[pl.ds(0, S)], rows_v, sem
            ).wait()

            def extract_grp(gi, _):
                gl = g_v[pl.ds(s0 + gi * _LANES, _LANES)]
                smp16 = gi * _LANES + lanes_iota
                r16 = gl % 8
                outbase = smp16 * _EMBED_DIM
                for c in range(_EMBED_DIM):
                    vals = plsc.load_gather(
                        rows_v, [smp16, r16, jnp.full((_LANES,), c, jnp.int32)])
                    plsc.store_scatter(xflat, [outbase + c], vals)
                return 0

            lax.fori_loop(0, S // _LANES, extract_grp, 0)
            pltpu.sync_copy(
                xflat,
                out_hbm.at[pl.ds((base + s0) * _EMBED_DIM, S * _EMBED_DIM)])
            return 0

        lax.fori_loop(0, n_stage, stage, 0)

    return gather_kernel(fi_flat, table3)


def _mlp_body(x_ref, w1_ref, b1_ref, w2_ref, b2_ref, w3_ref, b3_ref, o_ref):
    x = x_ref[...]
    h = jnp.dot(x, w1_ref[...], preferred_element_type=jnp.float32)
    h = jnp.maximum(h + b1_ref[...], 0.0)
    h = jnp.dot(h, w2_ref[...], preferred_element_type=jnp.float32)
    h = jnp.maximum(h + b2_ref[...], 0.0)
    o_ref[...] = jnp.sum(h * w3_ref[...], axis=1, keepdims=True) + b3_ref[...]


def _mlp(x, w1t, b1, w2t, b2, w3, b3):
    bsz, in_dim = x.shape
    blk = 512
    grid = bsz // blk
    return pl.pallas_call(
        _mlp_body,
        grid=(grid,),
        in_specs=[
            pl.BlockSpec((blk, in_dim), lambda i: (i, 0)),
            pl.BlockSpec(w1t.shape, lambda i: (0, 0)),
            pl.BlockSpec(b1.shape, lambda i: (0, 0)),
            pl.BlockSpec(w2t.shape, lambda i: (0, 0)),
            pl.BlockSpec(b2.shape, lambda i: (0, 0)),
            pl.BlockSpec(w3.shape, lambda i: (0, 0)),
            pl.BlockSpec(b3.shape, lambda i: (0, 0)),
        ],
        out_specs=pl.BlockSpec((blk, 1), lambda i: (i, 0)),
        out_shape=jax.ShapeDtypeStruct((bsz, 1), jnp.float32),
    )(x, w1t, b1, w2t, b2, w3, b3)


def kernel(field_indices, table, W1, b1, W2, b2, W3, b3):
    bsz, nf = field_indices.shape
    nc, ns = _sc_geometry()
    fi_flat = field_indices.reshape(-1).astype(jnp.int32)
    table3 = table.reshape(-1, 8, _EMBED_DIM)
    flat = _gather(fi_flat, table3, nc, ns)
    x = flat.reshape(bsz, nf * _EMBED_DIM)
    out = _mlp(x, W1.T, b1[None, :], W2.T, b2[None, :], W3, b3[None, :])
    return out[:, 0]


# speedup vs baseline: 4.3102x; 1.1634x over previous
import functools

import jax
import jax.numpy as jnp
from jax import lax
from jax.experimental import pallas as pl
from jax.experimental.pallas import tpu as pltpu
from jax.experimental.pallas import tpu_sc as plsc

_N_FIELDS = 26
_HASH_SIZE = 100000
_EMBED_DIM = 32
_LANES = 16


def _sc_geometry():
    try:
        info = plsc.get_sparse_core_info()
        return info.num_cores, info.num_subcores
    except Exception:
        return 2, 16


@functools.partial(jax.jit, static_argnums=(2, 3))
def _gather(fi_flat, table3, nc, ns):
    total = fi_flat.shape[0]
    nw = nc * ns
    per_w = total // nw
    S = 32
    n_stage = per_w // S
    n_super = n_stage // 8
    assert per_w % S == 0 and S % _LANES == 0 and per_w % _N_FIELDS == 0
    assert n_stage % 8 == 0

    mesh = plsc.VectorSubcoreMesh(core_axis_name="c", subcore_axis_name="s",
                                  num_cores=nc, num_subcores=ns)

    @functools.partial(
        pl.kernel,
        out_type=jax.ShapeDtypeStruct((total * _EMBED_DIM,), jnp.float32),
        mesh=mesh,
        scratch_types=[
            pltpu.VMEM((per_w,), jnp.int32),
            pltpu.VMEM((per_w,), jnp.int32),
            pltpu.VMEM((S, 8, _EMBED_DIM), jnp.float32),
            pltpu.VMEM((S, 8, _EMBED_DIM), jnp.float32),
            pltpu.VMEM((8 * S * _EMBED_DIM,), jnp.float32),
            pltpu.SemaphoreType.DMA,
            pltpu.SemaphoreType.DMA,
        ],
        compiler_params=pltpu.CompilerParams(needs_layout_passes=False),
    )
    def gather_kernel(fi_hbm, table_hbm, out_hbm, g_v, gq_v, rows_a, rows_b,
                      xacc, sem_a, sem_b):
        wid = lax.axis_index("s") * nc + lax.axis_index("c")
        base = wid * per_w
        lanes_iota = lax.iota(jnp.int32, _LANES)

        pltpu.sync_copy(fi_hbm.at[pl.ds(base, per_w)], g_v)

        def off(k, _):
            sl = pl.ds(k * _LANES, _LANES)
            pos = k * _LANES + lanes_iota
            g = g_v[sl] + (pos % _N_FIELDS) * _HASH_SIZE
            g_v[sl] = g
            gq_v[sl] = g // 8
            return 0

        lax.fori_loop(0, per_w // _LANES, off, 0)

        def fire(st, rows, sem):
            def fire_grp(gi, _):
                ql = gq_v[pl.ds(st * S + gi * _LANES, _LANES)]
                for l in range(_LANES):
                    q = jnp.sum(jnp.where(lanes_iota == l, ql, 0))
                    pltpu.make_async_copy(
                        table_hbm.at[q], rows.at[gi * _LANES + l], sem,
                    ).start()
                return 0

            lax.fori_loop(0, S // _LANES, fire_grp, 0)

        def drain(rows, sem):
            pltpu.make_async_copy(
                table_hbm.at[pl.ds(0, S)], rows, sem
            ).wait()

        def extract(st, rows, xoff):
            def extract_grp(gi, _):
                gl = g_v[pl.ds(st * S + gi * _LANES, _LANES)]
                smp16 = gi * _LANES + lanes_iota
                r16 = gl % 8
                outbase = xoff + smp16 * _EMBED_DIM
                for c in range(_EMBED_DIM):
                    vals = plsc.load_gather(
                        rows, [smp16, r16, jnp.full((_LANES,), c, jnp.int32)])
                    plsc.store_scatter(xacc, [outbase + c], vals)
                return 0

            lax.fori_loop(0, S // _LANES, extract_grp, 0)

        fire(0, rows_a, sem_a)

        def super_body(u, _):
            for p in range(4):
                e = u * 8 + 2 * p
                fire(e + 1, rows_b, sem_b)
                drain(rows_a, sem_a)
                extract(e, rows_a, (2 * p) * S * _EMBED_DIM)
                fire(jnp.minimum(e + 2, n_stage - 1), rows_a, sem_a)
                drain(rows_b, sem_b)
                extract(e + 1, rows_b, (2 * p + 1) * S * _EMBED_DIM)
            pltpu.sync_copy(
                xacc,
                out_hbm.at[pl.ds((base + u * 8 * S) * _EMBED_DIM,
                                 8 * S * _EMBED_DIM)])
            return 0

        lax.fori_loop(0, n_super, super_body, 0)
        drain(rows_a, sem_a)

    return gather_kernel(fi_flat, table3)


def _mlp_body(x_ref, w1_ref, b1_ref, w2_ref, b2_ref, w3_ref, b3_ref, o_ref):
    x = x_ref[...]
    h = jnp.dot(x, w1_ref[...], preferred_element_type=jnp.float32)
    h = jnp.maximum(h + b1_ref[...], 0.0)
    h = jnp.dot(h, w2_ref[...], preferred_element_type=jnp.float32)
    h = jnp.maximum(h + b2_ref[...], 0.0)
    o_ref[...] = jnp.sum(h * w3_ref[...], axis=1, keepdims=True) + b3_ref[...]


def _mlp(x, w1t, b1, w2t, b2, w3, b3):
    bsz, in_dim = x.shape
    blk = 512
    grid = bsz // blk
    return pl.pallas_call(
        _mlp_body,
        grid=(grid,),
        in_specs=[
            pl.BlockSpec((blk, in_dim), lambda i: (i, 0)),
            pl.BlockSpec(w1t.shape, lambda i: (0, 0)),
            pl.BlockSpec(b1.shape, lambda i: (0, 0)),
            pl.BlockSpec(w2t.shape, lambda i: (0, 0)),
            pl.BlockSpec(b2.shape, lambda i: (0, 0)),
            pl.BlockSpec(w3.shape, lambda i: (0, 0)),
            pl.BlockSpec(b3.shape, lambda i: (0, 0)),
        ],
        out_specs=pl.BlockSpec((blk, 1), lambda i: (i, 0)),
        out_shape=jax.ShapeDtypeStruct((bsz, 1), jnp.float32),
    )(x, w1t, b1, w2t, b2, w3, b3)


def kernel(field_indices, table, W1, b1, W2, b2, W3, b3):
    bsz, nf = field_indices.shape
    nc, ns = _sc_geometry()
    fi_flat = field_indices.reshape(-1).astype(jnp.int32)
    table3 = table.reshape(-1, 8, _EMBED_DIM)
    flat = _gather(fi_flat, table3, nc, ns)
    x = flat.reshape(bsz, nf * _EMBED_DIM)
    out = _mlp(x, W1.T, b1[None, :], W2.T, b2[None, :], W3, b3[None, :])
    return out[:, 0]


# speedup vs baseline: 4.4858x; 1.0408x over previous
import functools

import jax
import jax.numpy as jnp
from jax import lax
from jax.experimental import pallas as pl
from jax.experimental.pallas import tpu as pltpu
from jax.experimental.pallas import tpu_sc as plsc

_N_FIELDS = 26
_HASH_SIZE = 100000
_EMBED_DIM = 32
_LANES = 16


def _sc_geometry():
    try:
        info = plsc.get_sparse_core_info()
        return info.num_cores, info.num_subcores
    except Exception:
        return 2, 16


@functools.partial(jax.jit, static_argnums=(2, 3))
def _gather(fi_flat, table3, nc, ns):
    total = fi_flat.shape[0]
    nw = nc * ns
    per_w = total // nw
    S = 16
    n_stage = per_w // S
    n_super = n_stage // 8
    assert per_w % S == 0 and S % _LANES == 0 and per_w % _N_FIELDS == 0
    assert n_stage % 8 == 0

    mesh = plsc.VectorSubcoreMesh(core_axis_name="c", subcore_axis_name="s",
                                  num_cores=nc, num_subcores=ns)

    @functools.partial(
        pl.kernel,
        out_type=jax.ShapeDtypeStruct((total * _EMBED_DIM,), jnp.float32),
        mesh=mesh,
        scratch_types=[
            pltpu.VMEM((per_w,), jnp.int32),
            pltpu.VMEM((per_w,), jnp.int32),
            pltpu.VMEM((S, 8, _EMBED_DIM), jnp.float32),
            pltpu.VMEM((S, 8, _EMBED_DIM), jnp.float32),
            pltpu.VMEM((S, 8, _EMBED_DIM), jnp.float32),
            pltpu.VMEM((S, 8, _EMBED_DIM), jnp.float32),
            pltpu.VMEM((8 * S * _EMBED_DIM,), jnp.float32),
            pltpu.SemaphoreType.DMA,
            pltpu.SemaphoreType.DMA,
            pltpu.SemaphoreType.DMA,
            pltpu.SemaphoreType.DMA,
        ],
        compiler_params=pltpu.CompilerParams(needs_layout_passes=False),
    )
    def gather_kernel(fi_hbm, table_hbm, out_hbm, g_v, gq_v, rows_0, rows_1,
                      rows_2, rows_3, xacc, sem_0, sem_1, sem_2, sem_3):
        wid = lax.axis_index("s") * nc + lax.axis_index("c")
        base = wid * per_w
        lanes_iota = lax.iota(jnp.int32, _LANES)

        pltpu.sync_copy(fi_hbm.at[pl.ds(base, per_w)], g_v)

        def off(k, _):
            sl = pl.ds(k * _LANES, _LANES)
            pos = k * _LANES + lanes_iota
            g = g_v[sl] + (pos % _N_FIELDS) * _HASH_SIZE
            g_v[sl] = g
            gq_v[sl] = g // 8
            return 0

        lax.fori_loop(0, per_w // _LANES, off, 0)

        def fire(st, rows, sem):
            def fire_grp(gi, _):
                ql = gq_v[pl.ds(st * S + gi * _LANES, _LANES)]
                for l in range(_LANES):
                    q = jnp.sum(jnp.where(lanes_iota == l, ql, 0))
                    pltpu.make_async_copy(
                        table_hbm.at[q], rows.at[gi * _LANES + l], sem,
                    ).start()
                return 0

            lax.fori_loop(0, S // _LANES, fire_grp, 0)

        def drain(rows, sem):
            pltpu.make_async_copy(
                table_hbm.at[pl.ds(0, S)], rows, sem
            ).wait()

        def extract(st, rows, xoff):
            def extract_grp(gi, _):
                gl = g_v[pl.ds(st * S + gi * _LANES, _LANES)]
                smp16 = gi * _LANES + lanes_iota
                r16 = gl % 8
                outbase = xoff + smp16 * _EMBED_DIM
                for c in range(_EMBED_DIM):
                    vals = plsc.load_gather(
                        rows, [smp16, r16, jnp.full((_LANES,), c, jnp.int32)])
                    plsc.store_scatter(xacc, [outbase + c], vals)
                return 0

            lax.fori_loop(0, S // _LANES, extract_grp, 0)

        bufs = (rows_0, rows_1, rows_2, rows_3)
        sems = (sem_0, sem_1, sem_2, sem_3)
        fire(0, bufs[0], sems[0])
        fire(1, bufs[1], sems[1])

        def super_body(u, _):
            for p in range(8):
                st = u * 8 + p
                fire(jnp.minimum(st + 2, n_stage - 1),
                     bufs[(p + 2) % 4], sems[(p + 2) % 4])
                drain(bufs[p % 4], sems[p % 4])
                extract(st, bufs[p % 4], p * S * _EMBED_DIM)
            pltpu.sync_copy(
                xacc,
                out_hbm.at[pl.ds((base + u * 8 * S) * _EMBED_DIM,
                                 8 * S * _EMBED_DIM)])
            return 0

        lax.fori_loop(0, n_super, super_body, 0)
        drain(bufs[0], sems[0])
        drain(bufs[1], sems[1])

    return gather_kernel(fi_flat, table3)


def _mlp_body(x_ref, w1_ref, b1_ref, w2_ref, b2_ref, w3_ref, b3_ref, o_ref):
    x = x_ref[...]
    h = jnp.dot(x, w1_ref[...], preferred_element_type=jnp.float32)
    h = jnp.maximum(h + b1_ref[...], 0.0)
    h = jnp.dot(h, w2_ref[...], preferred_element_type=jnp.float32)
    h = jnp.maximum(h + b2_ref[...], 0.0)
    o_ref[...] = jnp.sum(h * w3_ref[...], axis=1, keepdims=True) + b3_ref[...]


def _mlp(x, w1t, b1, w2t, b2, w3, b3):
    bsz, in_dim = x.shape
    blk = 512
    grid = bsz // blk
    return pl.pallas_call(
        _mlp_body,
        grid=(grid,),
        in_specs=[
            pl.BlockSpec((blk, in_dim), lambda i: (i, 0)),
            pl.BlockSpec(w1t.shape, lambda i: (0, 0)),
            pl.BlockSpec(b1.shape, lambda i: (0, 0)),
            pl.BlockSpec(w2t.shape, lambda i: (0, 0)),
            pl.BlockSpec(b2.shape, lambda i: (0, 0)),
            pl.BlockSpec(w3.shape, lambda i: (0, 0)),
            pl.BlockSpec(b3.shape, lambda i: (0, 0)),
        ],
        out_specs=pl.BlockSpec((blk, 1), lambda i: (i, 0)),
        out_shape=jax.ShapeDtypeStruct((bsz, 1), jnp.float32),
    )(x, w1t, b1, w2t, b2, w3, b3)


def kernel(field_indices, table, W1, b1, W2, b2, W3, b3):
    bsz, nf = field_indices.shape
    nc, ns = _sc_geometry()
    fi_flat = field_indices.reshape(-1).astype(jnp.int32)
    table3 = table.reshape(-1, 8, _EMBED_DIM)
    flat = _gather(fi_flat, table3, nc, ns)
    x = flat.reshape(bsz, nf * _EMBED_DIM)
    out = _mlp(x, W1.T, b1[None, :], W2.T, b2[None, :], W3, b3[None, :])
    return out[:, 0]


# speedup vs baseline: 4.4881x; 1.0005x over previous
import functools

import jax
import jax.numpy as jnp
from jax import lax
from jax.experimental import pallas as pl
from jax.experimental.pallas import tpu as pltpu
from jax.experimental.pallas import tpu_sc as plsc

_N_FIELDS = 26
_HASH_SIZE = 100000
_EMBED_DIM = 32
_LANES = 16


def _sc_geometry():
    try:
        info = plsc.get_sparse_core_info()
        return info.num_cores, info.num_subcores
    except Exception:
        return 2, 16


@functools.partial(jax.jit, static_argnums=(2, 3))
def _gather(fi_flat, table3, nc, ns):
    total = fi_flat.shape[0]
    nw = nc * ns
    per_w = total // nw
    S = 16
    n_stage = per_w // S
    n_super = n_stage // 8
    assert per_w % S == 0 and S % _LANES == 0 and per_w % _N_FIELDS == 0
    assert n_stage % 8 == 0

    mesh = plsc.VectorSubcoreMesh(core_axis_name="c", subcore_axis_name="s",
                                  num_cores=nc, num_subcores=ns)

    @functools.partial(
        pl.kernel,
        out_type=jax.ShapeDtypeStruct((total * _EMBED_DIM,), jnp.float32),
        mesh=mesh,
        scratch_types=[
            pltpu.VMEM((per_w,), jnp.int32),
            pltpu.VMEM((per_w,), jnp.int32),
            pltpu.VMEM((S, 8, _EMBED_DIM), jnp.float32),
            pltpu.VMEM((S, 8, _EMBED_DIM), jnp.float32),
            pltpu.VMEM((S, 8, _EMBED_DIM), jnp.float32),
            pltpu.VMEM((S, 8, _EMBED_DIM), jnp.float32),
            pltpu.VMEM((8 * S * _EMBED_DIM,), jnp.float32),
            pltpu.SemaphoreType.DMA,
            pltpu.SemaphoreType.DMA,
            pltpu.SemaphoreType.DMA,
            pltpu.SemaphoreType.DMA,
        ],
        compiler_params=pltpu.CompilerParams(needs_layout_passes=False),
    )
    def gather_kernel(fi_hbm, table_hbm, out_hbm, g_v, gq_v, rows_0, rows_1,
                      rows_2, rows_3, xacc, sem_0, sem_1, sem_2, sem_3):
        wid = lax.axis_index("s") * nc + lax.axis_index("c")
        base = wid * per_w
        lanes_iota = lax.iota(jnp.int32, _LANES)

        pltpu.sync_copy(fi_hbm.at[pl.ds(base, per_w)], g_v)

        def off(k, _):
            sl = pl.ds(k * _LANES, _LANES)
            pos = k * _LANES + lanes_iota
            g = g_v[sl] + (pos % _N_FIELDS) * _HASH_SIZE
            g_v[sl] = g
            gq_v[sl] = g // 8
            return 0

        lax.fori_loop(0, per_w // _LANES, off, 0)

        def fire(st, rows, sem):
            for gi in range(S // _LANES):
                ql = gq_v[pl.ds(st * S + gi * _LANES, _LANES)]
                for l in range(_LANES):
                    q = jnp.sum(jnp.where(lanes_iota == l, ql, 0))
                    pltpu.make_async_copy(
                        table_hbm.at[q], rows.at[gi * _LANES + l], sem,
                    ).start()

        def drain(rows, sem):
            pltpu.make_async_copy(
                table_hbm.at[pl.ds(0, S)], rows, sem
            ).wait()

        def extract(st, rows, xoff):
            for gi in range(S // _LANES):
                gl = g_v[pl.ds(st * S + gi * _LANES, _LANES)]
                smp16 = gi * _LANES + lanes_iota
                r16 = gl % 8
                outbase = xoff + smp16 * _EMBED_DIM
                for c in range(_EMBED_DIM):
                    vals = plsc.load_gather(
                        rows, [smp16, r16, jnp.full((_LANES,), c, jnp.int32)])
                    plsc.store_scatter(xacc, [outbase + c], vals)

        bufs = (rows_0, rows_1, rows_2, rows_3)
        sems = (sem_0, sem_1, sem_2, sem_3)
        fire(0, bufs[0], sems[0])
        fire(1, bufs[1], sems[1])
        fire(2, bufs[2], sems[2])

        def super_body(u, _):
            for p in range(8):
                st = u * 8 + p
                fire(jnp.minimum(st + 3, n_stage - 1),
                     bufs[(p + 3) % 4], sems[(p + 3) % 4])
                drain(bufs[p % 4], sems[p % 4])
                extract(st, bufs[p % 4], p * S * _EMBED_DIM)
            pltpu.sync_copy(
                xacc,
                out_hbm.at[pl.ds((base + u * 8 * S) * _EMBED_DIM,
                                 8 * S * _EMBED_DIM)])
            return 0

        lax.fori_loop(0, n_super, super_body, 0)
        drain(bufs[0], sems[0])
        drain(bufs[1], sems[1])
        drain(bufs[2], sems[2])

    return gather_kernel(fi_flat, table3)


def _mlp_body(x_ref, w1_ref, b1_ref, w2_ref, b2_ref, w3_ref, b3_ref, o_ref):
    x = x_ref[...]
    h = jnp.dot(x, w1_ref[...], preferred_element_type=jnp.float32)
    h = jnp.maximum(h + b1_ref[...], 0.0)
    h = jnp.dot(h, w2_ref[...], preferred_element_type=jnp.float32)
    h = jnp.maximum(h + b2_ref[...], 0.0)
    o_ref[...] = jnp.sum(h * w3_ref[...], axis=1, keepdims=True) + b3_ref[...]


def _mlp(x, w1t, b1, w2t, b2, w3, b3):
    bsz, in_dim = x.shape
    blk = 512
    grid = bsz // blk
    return pl.pallas_call(
        _mlp_body,
        grid=(grid,),
        in_specs=[
            pl.BlockSpec((blk, in_dim), lambda i: (i, 0)),
            pl.BlockSpec(w1t.shape, lambda i: (0, 0)),
            pl.BlockSpec(b1.shape, lambda i: (0, 0)),
            pl.BlockSpec(w2t.shape, lambda i: (0, 0)),
            pl.BlockSpec(b2.shape, lambda i: (0, 0)),
            pl.BlockSpec(w3.shape, lambda i: (0, 0)),
            pl.BlockSpec(b3.shape, lambda i: (0, 0)),
        ],
        out_specs=pl.BlockSpec((blk, 1), lambda i: (i, 0)),
        out_shape=jax.ShapeDtypeStruct((bsz, 1), jnp.float32),
    )(x, w1t, b1, w2t, b2, w3, b3)


def kernel(field_indices, table, W1, b1, W2, b2, W3, b3):
    bsz, nf = field_indices.shape
    nc, ns = _sc_geometry()
    fi_flat = field_indices.reshape(-1).astype(jnp.int32)
    table3 = table.reshape(-1, 8, _EMBED_DIM)
    flat = _gather(fi_flat, table3, nc, ns)
    x = flat.reshape(bsz, nf * _EMBED_DIM)
    out = _mlp(x, W1.T, b1[None, :], W2.T, b2[None, :], W3, b3[None, :])
    return out[:, 0]
